# baseline (speedup 1.0000x reference)
import functools

import jax
import jax.numpy as jnp
from jax import lax
from jax.experimental import pallas as pl
from jax.experimental.pallas import tpu as pltpu
from jax.experimental.pallas import tpu_sc as plsc

NUM_CORES = 2
NUM_SUBCORES = 16
NUM_WORKERS = NUM_CORES * NUM_SUBCORES

ROWS_PER_CHUNK = 8
IDX_SPLIT = 128


def _emb_kernel(n_chunks, s, v, d, table_hbm, idx_hbm, out_hbm,
                table_sh, idx_v, rows_v, sem):
    cid = lax.axis_index("c")
    sid = lax.axis_index("s")
    wid = sid * NUM_CORES + cid
    chunk = ROWS_PER_CHUNK * s

    @pl.when(sid == 0)
    def _():
        pltpu.sync_copy(table_hbm, table_sh)

    plsc.subcore_barrier()

    def chunk_body(c, carry):
        r0 = pl.multiple_of((wid * n_chunks + c) * ROWS_PER_CHUNK, 8)
        pltpu.sync_copy(idx_hbm.at[pl.ds(r0, ROWS_PER_CHUNK)], idx_v)
        copies = []
        for j in range(ROWS_PER_CHUNK):
            copies.append(pltpu.async_copy(
                table_sh.at[idx_v.at[j, pl.ds(0, IDX_SPLIT)]],
                rows_v.at[pl.ds(j * s, IDX_SPLIT)],
                sem,
            ))
            copies.append(pltpu.async_copy(
                table_sh.at[idx_v.at[j, pl.ds(IDX_SPLIT, s - IDX_SPLIT)]],
                rows_v.at[pl.ds(j * s + IDX_SPLIT, s - IDX_SPLIT)],
                sem,
            ))
        for cp in copies:
            cp.wait()
        pltpu.sync_copy(rows_v, out_hbm.at[pl.ds(r0 * s, chunk)])
        return carry

    lax.fori_loop(0, n_chunks, chunk_body, 0)


def kernel(x, blosum):
    b0, s = x.shape
    v, d = blosum.shape
    b = b0 * s
    assert b0 % (NUM_WORKERS * ROWS_PER_CHUNK) == 0
    n_chunks = b0 // (NUM_WORKERS * ROWS_PER_CHUNK)

    idx = x.astype(jnp.int32)

    mesh = plsc.VectorSubcoreMesh(
        core_axis_name="c", subcore_axis_name="s",
        num_cores=NUM_CORES, num_subcores=NUM_SUBCORES,
    )
    emb = pl.kernel(
        functools.partial(_emb_kernel, n_chunks, s, v, d),
        out_type=jax.ShapeDtypeStruct((b, d), jnp.float32),
        mesh=mesh,
        scratch_types=[
            pltpu.VMEM_SHARED((v, d), jnp.float32),
            pltpu.VMEM((ROWS_PER_CHUNK, s), jnp.int32),
            pltpu.VMEM((ROWS_PER_CHUNK * s, d), jnp.float32),
            pltpu.SemaphoreType.DMA,
        ],
        compiler_params=pltpu.CompilerParams(use_tc_tiling_on_sc=False),
    )
    out = emb(blosum, idx)
    return out.reshape(b0, s, d)


# speedup vs baseline: 1.9222x; 1.9222x over previous
import functools

import jax
import jax.numpy as jnp
from jax import lax
from jax.experimental import pallas as pl
from jax.experimental.pallas import tpu as pltpu
from jax.experimental.pallas import tpu_sc as plsc

NUM_CORES = 2
NUM_SUBCORES = 16
NUM_WORKERS = NUM_CORES * NUM_SUBCORES

LANES = 16
TILE_I = 128
TILE_K = 8
NJ = 8


def _emb_kernel(n_it, s, v, d, table_hbm, x_hbm, out_hbm,
                table_v, idx_v, to_v, sem):
    cid = lax.axis_index("c")
    sid = lax.axis_index("s")
    wid = sid * NUM_CORES + cid
    chunk_words = TILE_I * s

    pltpu.sync_copy(table_hbm, table_v)

    il_vecs = [(jnp.arange(LANES, dtype=jnp.int32) + v8 * LANES) * s
               for v8 in range(TILE_I // LANES)]

    def chunk_body(c, carry):
        iti = wid * n_it + c
        base = pl.multiple_of(iti * chunk_words, 8)
        pltpu.sync_copy(x_hbm.at[pl.ds(base, chunk_words)], idx_v)

        def j_body(j, carry2):
            j_l = lax.rem(j, NJ)
            g = lax.div(j, NJ)
            buf = lax.rem(g, 2)

            @pl.when(jnp.logical_and(j_l == 0, g >= 2))
            def _():
                pltpu.make_async_copy(
                    to_v.at[0],
                    out_hbm.at[pl.ds(0, NJ), :, 0],
                    sem,
                ).wait()

            for v8 in range(TILE_I // LANES):
                xv = plsc.load_gather(idx_v, [il_vecs[v8] + j])
                xvd = xv * d
                for k in range(d):
                    vals = plsc.load_gather(table_v, [xvd + k])
                    to_v[buf, j_l, k // TILE_K, k % TILE_K,
                         pl.ds(v8 * LANES, LANES)] = vals

            @pl.when(j_l == NJ - 1)
            def _():
                j0 = pl.multiple_of(j - (NJ - 1), NJ)
                pltpu.async_copy(
                    to_v.at[buf],
                    out_hbm.at[pl.ds(j0, NJ), :, iti],
                    sem,
                )
            return carry2

        lax.fori_loop(0, s, j_body, 0)

        for _ in range(2):
            pltpu.make_async_copy(
                to_v.at[0],
                out_hbm.at[pl.ds(0, NJ), :, 0],
                sem,
            ).wait()
        return carry

    lax.fori_loop(0, n_it, chunk_body, 0)


def kernel(x, blosum):
    b0, s = x.shape
    v, d = blosum.shape
    assert b0 % (NUM_WORKERS * TILE_I) == 0
    assert d % TILE_K == 0 and s % NJ == 0
    n_it = b0 // (NUM_WORKERS * TILE_I)
    kt_n = d // TILE_K

    idx = x.reshape(b0 * s).astype(jnp.int32)
    table = blosum.reshape(v * d)

    mesh = plsc.VectorSubcoreMesh(
        core_axis_name="c", subcore_axis_name="s",
        num_cores=NUM_CORES, num_subcores=NUM_SUBCORES,
    )
    emb = pl.kernel(
        functools.partial(_emb_kernel, n_it, s, v, d),
        out_type=jax.ShapeDtypeStruct(
            (s, kt_n, b0 // TILE_I, TILE_K, TILE_I), jnp.float32),
        mesh=mesh,
        scratch_types=[
            pltpu.VMEM((v * d,), jnp.float32),
            pltpu.VMEM((TILE_I * s,), jnp.int32),
            pltpu.VMEM((2, NJ, kt_n, TILE_K, TILE_I), jnp.float32),
            pltpu.SemaphoreType.DMA,
        ],
        compiler_params=pltpu.CompilerParams(
            use_tc_tiling_on_sc=False, needs_layout_passes=False),
    )
    out5 = emb(table, idx)
    return out5.transpose(2, 4, 0, 1, 3).reshape(b0, s, d)


# speedup vs baseline: 6.4927x; 3.3777x over previous
import functools

import jax
import jax.numpy as jnp
from jax import lax
from jax.experimental import pallas as pl
from jax.experimental.pallas import tpu as pltpu
from jax.experimental.pallas import tpu_sc as plsc

NUM_CORES = 2
NUM_SUBCORES = 16
NUM_WORKERS = NUM_CORES * NUM_SUBCORES

LANES = 16
TILE_I = 128
TILE_K = 8
NJ = 8


def _emb_kernel(n_it, s, v, d, table_hbm, x_hbm, out_hbm,
                table_v, idx_v, to_v, sem):
    cid = lax.axis_index("c")
    sid = lax.axis_index("s")
    wid = sid * NUM_CORES + cid
    chunk_words = TILE_I * s

    pltpu.sync_copy(table_hbm, table_v)

    il_vecs = [(jnp.arange(LANES, dtype=jnp.int32) + v8 * LANES) * s
               for v8 in range(TILE_I // LANES)]

    def chunk_body(c, carry):
        iti = wid * n_it + c
        base = pl.multiple_of(iti * chunk_words, 8)
        pltpu.sync_copy(x_hbm.at[pl.ds(base, chunk_words)], idx_v)

        def j_body(j, carry2):
            j_l = lax.rem(j, NJ)
            g = lax.div(j, NJ)
            buf = lax.rem(g, 2)

            @pl.when(jnp.logical_and(j_l == 0, g >= 2))
            def _():
                pltpu.make_async_copy(
                    to_v.at[0],
                    out_hbm.at[pl.ds(0, NJ), :, 0],
                    sem,
                ).wait()

            for v8 in range(TILE_I // LANES):
                xv = plsc.load_gather(idx_v, [il_vecs[v8] + j])
                xvd = xv * d
                vals = [plsc.load_gather(table_v, [xvd + k])
                        for k in range(d)]
                for k in range(d):
                    to_v[buf, j_l, k // TILE_K, k % TILE_K,
                         pl.ds(v8 * LANES, LANES)] = vals[k]

            @pl.when(j_l == NJ - 1)
            def _():
                j0 = pl.multiple_of(j - (NJ - 1), NJ)
                pltpu.async_copy(
                    to_v.at[buf],
                    out_hbm.at[pl.ds(j0, NJ), :, iti],
                    sem,
                )
            return carry2

        lax.fori_loop(0, s, j_body, 0)

        for _ in range(2):
            pltpu.make_async_copy(
                to_v.at[0],
                out_hbm.at[pl.ds(0, NJ), :, 0],
                sem,
            ).wait()
        return carry

    lax.fori_loop(0, n_it, chunk_body, 0)


def kernel(x, blosum):
    b0, s = x.shape
    v, d = blosum.shape
    assert b0 % (NUM_WORKERS * TILE_I) == 0
    assert d % TILE_K == 0 and s % NJ == 0
    n_it = b0 // (NUM_WORKERS * TILE_I)
    kt_n = d // TILE_K

    idx = x.reshape(b0 * s).astype(jnp.int32)
    table = blosum.reshape(v * d)

    mesh = plsc.VectorSubcoreMesh(
        core_axis_name="c", subcore_axis_name="s",
        num_cores=NUM_CORES, num_subcores=NUM_SUBCORES,
    )
    emb = pl.kernel(
        functools.partial(_emb_kernel, n_it, s, v, d),
        out_type=jax.ShapeDtypeStruct(
            (s, kt_n, b0 // TILE_I, TILE_K, TILE_I), jnp.float32),
        mesh=mesh,
        scratch_types=[
            pltpu.VMEM((v * d,), jnp.float32),
            pltpu.VMEM((TILE_I * s,), jnp.int32),
            pltpu.VMEM((2, NJ, kt_n, TILE_K, TILE_I), jnp.float32),
            pltpu.SemaphoreType.DMA,
        ],
        compiler_params=pltpu.CompilerParams(
            use_tc_tiling_on_sc=False, needs_layout_passes=False),
    )
    out5 = emb(table, idx)
    return out5.transpose(2, 4, 0, 1, 3).reshape(b0, s, d)
